# baseline (speedup 1.0000x reference)
import jax
import jax.numpy as jnp
from jax.experimental import pallas as pl
from jax.experimental.pallas import tpu as pltpu

_S, _D, _U, _E, _K = 256, 256, 512, 64, 2
_G = 8
_NG = _E // _G


def _routing_kernel(x_ref, wr_ref, wsel_ref):
    x = x_ref[...]
    wr = wr_ref[...]
    logits = jax.lax.dot_general(
        x, wr, (((1,), (1,)), ((), ())), preferred_element_type=jnp.float32
    )
    e_iota = jax.lax.broadcasted_iota(jnp.int32, logits.shape, 1)
    i1 = jnp.argmax(logits, axis=1)
    m1 = jnp.max(logits, axis=1, keepdims=True)
    masked = jnp.where(e_iota == i1[:, None], -jnp.inf, logits)
    i2 = jnp.argmax(masked, axis=1)
    m2 = jnp.max(masked, axis=1, keepdims=True)
    w1 = jax.nn.sigmoid(m1 - m2)
    w2 = 1.0 - w1
    wsel = jnp.where(e_iota == i1[:, None], w1, 0.0) + jnp.where(
        e_iota == i2[:, None], w2, 0.0
    )
    wsel_ref[...] = wsel


def _expert_kernel(x_ref, wsel_ref, wu_ref, wd_ref, bu_ref, bd_ref, out_ref):
    g = pl.program_id(0)
    x = x_ref[...]
    e_iota = jax.lax.broadcasted_iota(jnp.int32, wsel_ref.shape, 1)
    acc = None
    for j in range(_G):
        h = jax.lax.dot_general(
            x.astype(jnp.bfloat16), wu_ref[j].astype(jnp.bfloat16),
            (((1,), (1,)), ((), ())),
            preferred_element_type=jnp.float32,
        )
        h = h + bu_ref[j]
        h = 0.5 * h * (1.0 + jax.lax.erf(h * 0.7071067811865476))
        y = jax.lax.dot_general(
            h.astype(jnp.bfloat16), wd_ref[j].astype(jnp.bfloat16),
            (((1,), (1,)), ((), ())),
            preferred_element_type=jnp.float32,
        )
        y = y + bd_ref[j]
        wcol = jnp.sum(
            jnp.where(e_iota == g * _G + j, wsel_ref[...], 0.0),
            axis=1, keepdims=True,
        )
        contrib = y * wcol
        acc = contrib if acc is None else acc + contrib

    @pl.when(g == 0)
    def _init():
        out_ref[...] = acc

    @pl.when(g != 0)
    def _acc():
        out_ref[...] += acc


def kernel(x, W_router, W_up, W_down, b_up, b_down):
    b, s, d = x.shape
    x2 = x.reshape(s, d)

    wsel = pl.pallas_call(
        _routing_kernel,
        out_shape=jax.ShapeDtypeStruct((_S, _E), jnp.float32),
    )(x2, W_router)

    bu3 = b_up.reshape(_E, 1, _U)
    bd3 = b_down.reshape(_E, 1, _D)

    out = pl.pallas_call(
        _expert_kernel,
        grid=(_NG,),
        in_specs=[
            pl.BlockSpec((_S, _D), lambda g: (0, 0)),
            pl.BlockSpec((_S, _E), lambda g: (0, 0)),
            pl.BlockSpec((_G, _U, _D), lambda g: (g, 0, 0)),
            pl.BlockSpec((_G, _D, _U), lambda g: (g, 0, 0)),
            pl.BlockSpec((_G, 1, _U), lambda g: (g, 0, 0)),
            pl.BlockSpec((_G, 1, _D), lambda g: (g, 0, 0)),
        ],
        out_specs=pl.BlockSpec((_S, _D), lambda g: (0, 0)),
        out_shape=jax.ShapeDtypeStruct((_S, _D), jnp.float32),
        compiler_params=pltpu.CompilerParams(
            dimension_semantics=("arbitrary",),
        ),
    )(x2, wsel, W_up, W_down, bu3, bd3)

    return out.reshape(b, s, d)


# speedup vs baseline: 1.0620x; 1.0620x over previous
import jax
import jax.numpy as jnp
from jax.experimental import pallas as pl
from jax.experimental.pallas import tpu as pltpu

_S, _D, _U, _E, _K = 256, 256, 512, 64, 2
_G = 8
_NG = _E // _G


def _moe_kernel(x_ref, wr_ref, wu_ref, wd_ref, bu_ref, bd_ref, out_ref,
                wsel_ref):
    g = pl.program_id(0)
    x = x_ref[...]

    @pl.when(g == 0)
    def _route():
        wr = wr_ref[...]
        logits = jax.lax.dot_general(
            x, wr, (((1,), (1,)), ((), ())),
            preferred_element_type=jnp.float32,
        )
        e_iota = jax.lax.broadcasted_iota(jnp.int32, logits.shape, 1)
        i1 = jnp.argmax(logits, axis=1)
        m1 = jnp.max(logits, axis=1, keepdims=True)
        masked = jnp.where(e_iota == i1[:, None], -jnp.inf, logits)
        i2 = jnp.argmax(masked, axis=1)
        m2 = jnp.max(masked, axis=1, keepdims=True)
        w1 = jax.nn.sigmoid(m1 - m2)
        w2 = 1.0 - w1
        wsel_ref[...] = jnp.where(e_iota == i1[:, None], w1, 0.0) + jnp.where(
            e_iota == i2[:, None], w2, 0.0
        )

    e_iota = jax.lax.broadcasted_iota(jnp.int32, wsel_ref.shape, 1)
    acc = None
    for j in range(_G):
        h = jax.lax.dot_general(
            x, wu_ref[j], (((1,), (1,)), ((), ())),
            preferred_element_type=jnp.float32,
        )
        h = h + bu_ref[j]
        h = 0.5 * h * (1.0 + jax.lax.erf(h * 0.7071067811865476))
        y = jax.lax.dot_general(
            h, wd_ref[j], (((1,), (1,)), ((), ())),
            preferred_element_type=jnp.float32,
        )
        y = y + bd_ref[j]
        wcol = jnp.sum(
            jnp.where(e_iota == g * _G + j, wsel_ref[...], 0.0),
            axis=1, keepdims=True,
        )
        contrib = y * wcol
        acc = contrib if acc is None else acc + contrib

    @pl.when(g == 0)
    def _init():
        out_ref[...] = acc

    @pl.when(g != 0)
    def _acc():
        out_ref[...] += acc


def kernel(x, W_router, W_up, W_down, b_up, b_down):
    b, s, d = x.shape
    x2 = x.reshape(s, d)

    bu3 = b_up.reshape(_E, 1, _U)
    bd3 = b_down.reshape(_E, 1, _D)

    out = pl.pallas_call(
        _moe_kernel,
        grid=(_NG,),
        in_specs=[
            pl.BlockSpec((_S, _D), lambda g: (0, 0)),
            pl.BlockSpec((_E, _D), lambda g: (0, 0)),
            pl.BlockSpec((_G, _U, _D), lambda g: (g, 0, 0)),
            pl.BlockSpec((_G, _D, _U), lambda g: (g, 0, 0)),
            pl.BlockSpec((_G, 1, _U), lambda g: (g, 0, 0)),
            pl.BlockSpec((_G, 1, _D), lambda g: (g, 0, 0)),
        ],
        out_specs=pl.BlockSpec((_S, _D), lambda g: (0, 0)),
        out_shape=jax.ShapeDtypeStruct((_S, _D), jnp.float32),
        scratch_shapes=[pltpu.VMEM((_S, _E), jnp.float32)],
        compiler_params=pltpu.CompilerParams(
            dimension_semantics=("arbitrary",),
        ),
    )(x2, W_router, W_up, W_down, bu3, bd3)

    return out.reshape(b, s, d)
